# Initial kernel scaffold; baseline (speedup 1.0000x reference)
import functools

import jax
import jax.numpy as jnp
from jax import lax
from jax.experimental import pallas as pl
from jax.experimental.pallas import tpu as pltpu
from jax.experimental.pallas import tpu_sc as plsc

NC = 2
NS = 16
NW = NC * NS
LANES = 16

CE = 6400
KB = CE // 128
EC = 8
PER_W = EC * CE
E_PAD = NW * PER_W

CN = 3200
KN = CN // 128
N_PAD = NW * CN


def _sc_mesh():
    return plsc.VectorSubcoreMesh(core_axis_name="c", subcore_axis_name="s",
                                  num_cores=NC)


@functools.lru_cache(maxsize=None)
def _make_gather_rows(nt, d, nchunks, ce):
    kb = ce // 128

    @functools.partial(
        pl.kernel, mesh=_sc_mesh(),
        out_type=jax.ShapeDtypeStruct((NW * nchunks * ce, d), jnp.float32),
        scratch_types=[
            pltpu.VMEM((kb, 128), jnp.int32),
            pltpu.VMEM((ce, d), jnp.float32),
            pltpu.SemaphoreType.DMA,
        ],
    )
    def k(table_hbm, idx_hbm, out_hbm, idx_v, rows_v, sem):
        w = lax.axis_index("s") * NC + lax.axis_index("c")

        def chunk(j, carry):
            pltpu.sync_copy(idx_hbm.at[w, j], idx_v)
            handles = []
            for b in range(kb):
                handles.append(pltpu.async_copy(
                    table_hbm.at[idx_v.at[b]],
                    rows_v.at[pl.ds(b * 128, 128)], sem))
            for h in handles:
                h.wait()
            pltpu.sync_copy(rows_v,
                            out_hbm.at[pl.ds((w * nchunks + j) * ce, ce)])
            return carry

        lax.fori_loop(0, nchunks, chunk, 0)

    return k


@functools.lru_cache(maxsize=None)
def _make_gather_scalar(nt, nchunks, ce):
    kb = ce // 128

    @functools.partial(
        pl.kernel, mesh=_sc_mesh(),
        out_type=jax.ShapeDtypeStruct((NW * nchunks * ce,), jnp.float32),
        scratch_types=[
            pltpu.VMEM((nt,), jnp.float32),
            pltpu.VMEM((kb, 128), jnp.int32),
            pltpu.VMEM((ce,), jnp.float32),
        ],
    )
    def k(table_hbm, idx_hbm, out_hbm, tab_v, idx_v, out_v):
        w = lax.axis_index("s") * NC + lax.axis_index("c")
        pltpu.sync_copy(table_hbm, tab_v)

        def chunk(j, carry):
            pltpu.sync_copy(idx_hbm.at[w, j], idx_v)

            def step(i, c2):
                row = i // 8
                off = (i % 8) * 16
                ivec = idx_v[row, pl.ds(off, 16)]
                out_v[pl.ds(i * 16, 16)] = plsc.load_gather(tab_v, [ivec])
                return c2

            lax.fori_loop(0, ce // 16, step, 0)
            pltpu.sync_copy(out_v,
                            out_hbm.at[pl.ds((w * nchunks + j) * ce, ce)])
            return carry

        lax.fori_loop(0, nchunks, chunk, 0)

    return k


@functools.lru_cache(maxsize=None)
def _make_scatter_rows(nt_pad, d, nchunks, ce):
    kb = ce // 128
    stripe = nt_pad // NS

    @functools.partial(
        pl.kernel, mesh=_sc_mesh(),
        out_type=jax.ShapeDtypeStruct((NC, nt_pad, d), jnp.float32),
        scratch_types=[
            pltpu.VMEM_SHARED((nt_pad, d), jnp.float32),
            pltpu.VMEM((kb, 128), jnp.int32),
            pltpu.VMEM((ce, d), jnp.float32),
            pltpu.SemaphoreType.DMA,
        ],
    )
    def k(vals_hbm, idx_hbm, out_hbm, acc, idx_v, vals_v, sem):
        c = lax.axis_index("c")
        s = lax.axis_index("s")
        w = s * NC + c

        zero = jnp.zeros((16,), jnp.float32)

        def zrow(i, carry):
            vals_v[i] = zero
            return carry

        lax.fori_loop(0, stripe, zrow, 0)
        pltpu.sync_copy(vals_v.at[pl.ds(0, stripe)],
                        acc.at[pl.ds(s * stripe, stripe)])
        plsc.subcore_barrier()

        def chunk(j, carry):
            pltpu.sync_copy(idx_hbm.at[w, j], idx_v)
            pltpu.sync_copy(vals_hbm.at[w, j], vals_v)
            handles = []
            for b in range(kb):
                handles.append(pltpu.async_copy(
                    vals_v.at[pl.ds(b * 128, 128)],
                    acc.at[idx_v.at[b]], sem, add=True))
            for h in handles:
                h.wait()
            return carry

        lax.fori_loop(0, nchunks, chunk, 0)
        plsc.subcore_barrier()
        pltpu.sync_copy(acc.at[pl.ds(s * stripe, stripe)],
                        out_hbm.at[c, pl.ds(s * stripe, stripe)])

    return k


@functools.lru_cache(maxsize=None)
def _make_scatter_scalar(nt_pad, nchunks, ce):
    kb = ce // 128
    stripe = nt_pad // NS

    @functools.partial(
        pl.kernel, mesh=_sc_mesh(),
        out_type=jax.ShapeDtypeStruct((NC, nt_pad), jnp.float32),
        scratch_types=[
            pltpu.VMEM_SHARED((nt_pad,), jnp.float32),
            pltpu.VMEM((kb, 128), jnp.int32),
            pltpu.VMEM((ce,), jnp.float32),
            pltpu.SemaphoreType.DMA,
        ],
    )
    def k(vals_hbm, idx_hbm, out_hbm, acc, idx_v, vals_v, sem):
        c = lax.axis_index("c")
        s = lax.axis_index("s")
        w = s * NC + c

        zero = jnp.zeros((16,), jnp.float32)

        def zrow(i, carry):
            vals_v[pl.ds(i * 16, 16)] = zero
            return carry

        lax.fori_loop(0, stripe // 16, zrow, 0)
        pltpu.sync_copy(vals_v.at[pl.ds(0, stripe)],
                        acc.at[pl.ds(s * stripe, stripe)])
        plsc.subcore_barrier()

        def chunk(j, carry):
            pltpu.sync_copy(idx_hbm.at[w, j], idx_v)
            pltpu.sync_copy(vals_hbm.at[w, j], vals_v)
            handles = []
            for b in range(kb):
                handles.append(pltpu.async_copy(
                    vals_v.at[pl.ds(b * 128, 128)],
                    acc.at[idx_v.at[b]], sem, add=True))
            for h in handles:
                h.wait()
            return carry

        lax.fori_loop(0, nchunks, chunk, 0)
        plsc.subcore_barrier()
        pltpu.sync_copy(acc.at[pl.ds(s * stripe, stripe)],
                        out_hbm.at[c, pl.ds(s * stripe, stripe)])

    return k


def _gather_rows(table, idx4):
    nw, nchunks, kb, _ = idx4.shape
    return _make_gather_rows(table.shape[0], table.shape[1], nchunks,
                             kb * 128)(table, idx4)


def _gather_scalar(table, idx4):
    nw, nchunks, kb, _ = idx4.shape
    return _make_gather_scalar(table.shape[0], nchunks, kb * 128)(table, idx4)


def _scatter_rows(vals3, idx4, nt_pad):
    nw, nchunks, ce, d = vals3.shape
    return _make_scatter_rows(nt_pad, d, nchunks, ce)(vals3, idx4)


def _scatter_scalar(vals3, idx4, nt_pad):
    nw, nchunks, ce = vals3.shape
    return _make_scatter_scalar(nt_pad, nchunks, ce)(vals3, idx4)


_NROW = 1000


def _mm_attn(h, W, a2):
    n, din = h.shape
    dout = W.shape[1]

    def body(h_ref, w_ref, a_ref, hw_ref, esd_ref):
        hw = jnp.dot(h_ref[...], w_ref[...], preferred_element_type=jnp.float32)
        hw_ref[...] = hw
        esd_ref[...] = jnp.dot(hw, a_ref[...], preferred_element_type=jnp.float32)

    return pl.pallas_call(
        body,
        grid=(n // _NROW,),
        in_specs=[
            pl.BlockSpec((_NROW, din), lambda i: (i, 0)),
            pl.BlockSpec((din, dout), lambda i: (0, 0)),
            pl.BlockSpec((dout, 2), lambda i: (0, 0)),
        ],
        out_specs=[
            pl.BlockSpec((_NROW, dout), lambda i: (i, 0)),
            pl.BlockSpec((_NROW, 2), lambda i: (i, 0)),
        ],
        out_shape=[
            jax.ShapeDtypeStruct((n, dout), jnp.float32),
            jax.ShapeDtypeStruct((n, 2), jnp.float32),
        ],
    )(h, W, a2)


def _ew_grid(m, bm=12800):
    assert m % bm == 0, (m, bm)
    return m // bm, bm


def _leaky(t1, t2):
    g, bm = _ew_grid(t1.shape[0])

    def body(a_ref, b_ref, o_ref):
        s = a_ref[...] + b_ref[...]
        o_ref[...] = jnp.where(s >= 0, s, 0.2 * s)

    return pl.pallas_call(
        body, grid=(g,),
        in_specs=[pl.BlockSpec((bm,), lambda i: (i,))] * 2,
        out_specs=pl.BlockSpec((bm,), lambda i: (i,)),
        out_shape=jax.ShapeDtypeStruct(t1.shape, jnp.float32),
    )(t1, t2)


def _gmax(e):
    g, bm = _ew_grid(e.shape[0])

    def body(e_ref, o_ref):
        i = pl.program_id(0)

        @pl.when(i == 0)
        def _():
            o_ref[...] = jnp.full((1, 1), -jnp.inf, jnp.float32)

        o_ref[...] = jnp.maximum(o_ref[...], jnp.max(e_ref[...]))

    return pl.pallas_call(
        body, grid=(g,),
        in_specs=[pl.BlockSpec((bm,), lambda i: (i,))],
        out_specs=pl.BlockSpec((1, 1), lambda i: (0, 0)),
        out_shape=jax.ShapeDtypeStruct((1, 1), jnp.float32),
    )(e)


def _exp_shift_mask(e, gm, mask):
    g, bm = _ew_grid(e.shape[0])

    def body(e_ref, g_ref, m_ref, o_ref):
        o_ref[...] = jnp.exp(e_ref[...] - g_ref[0, 0]) * m_ref[...]

    return pl.pallas_call(
        body, grid=(g,),
        in_specs=[
            pl.BlockSpec((bm,), lambda i: (i,)),
            pl.BlockSpec((1, 1), lambda i: (0, 0)),
            pl.BlockSpec((bm,), lambda i: (i,)),
        ],
        out_specs=pl.BlockSpec((bm,), lambda i: (i,)),
        out_shape=jax.ShapeDtypeStruct(e.shape, jnp.float32),
    )(e, gm, mask)


def _mul(a, b):
    g, bm = _ew_grid(a.shape[0])

    def body(a_ref, b_ref, o_ref):
        o_ref[...] = a_ref[...] * b_ref[...]

    return pl.pallas_call(
        body, grid=(g,),
        in_specs=[pl.BlockSpec((bm,), lambda i: (i,))] * 2,
        out_specs=pl.BlockSpec((bm,), lambda i: (i,)),
        out_shape=jax.ShapeDtypeStruct(a.shape, jnp.float32),
    )(a, b)


def _gl_fuse(r, alpha, s_e, e):
    g, bm = _ew_grid(r.shape[0])

    def body(r_ref, a_ref, s_ref, e_ref, o_ref):
        ge = r_ref[...] - a_ref[...] * s_ref[...]
        o_ref[...] = ge * jnp.where(e_ref[...] >= 0, 1.0, 0.2)

    return pl.pallas_call(
        body, grid=(g,),
        in_specs=[pl.BlockSpec((bm,), lambda i: (i,))] * 4,
        out_specs=pl.BlockSpec((bm,), lambda i: (i,)),
        out_shape=jax.ShapeDtypeStruct(r.shape, jnp.float32),
    )(r, alpha, s_e, e)


def _scale_rows(rows, s):
    m, d = rows.shape
    bm = 12800
    assert m % bm == 0

    def body(r_ref, s_ref, o_ref):
        o_ref[...] = r_ref[...] * s_ref[...]

    return pl.pallas_call(
        body, grid=(m // bm,),
        in_specs=[
            pl.BlockSpec((bm, d), lambda i: (i, 0)),
            pl.BlockSpec((bm, 1), lambda i: (i, 0)),
        ],
        out_specs=pl.BlockSpec((bm, d), lambda i: (i, 0)),
        out_shape=jax.ShapeDtypeStruct((m, d), jnp.float32),
    )(rows, s)


def _dot_rows(a, b):
    m, d = a.shape
    bm = 12800
    assert m % bm == 0

    def body(a_ref, b_ref, o_ref):
        o_ref[...] = jnp.sum(a_ref[...] * b_ref[...], axis=-1)

    return pl.pallas_call(
        body, grid=(m // bm,),
        in_specs=[pl.BlockSpec((bm, d), lambda i: (i, 0))] * 2,
        out_specs=pl.BlockSpec((bm,), lambda i: (i,)),
        out_shape=jax.ShapeDtypeStruct((m,), jnp.float32),
    )(a, b)


def _pick_bm(m):
    for bm in (16384, 12800, 12544, 8192, 4096, 2048, 1024, 512, 256, 128):
        if m % bm == 0:
            return bm
    raise ValueError(m)


def _sum_leading(x):
    k, m = x.shape
    bm = _pick_bm(m)

    def body(x_ref, o_ref):
        o_ref[...] = jnp.sum(x_ref[...], axis=0)

    return pl.pallas_call(
        body, grid=(m // bm,),
        in_specs=[pl.BlockSpec((k, bm), lambda i: (0, i))],
        out_specs=pl.BlockSpec((bm,), lambda i: (i,)),
        out_shape=jax.ShapeDtypeStruct((m,), jnp.float32),
    )(x)


def _inv_clip(x, lo):
    k, m = x.shape
    bm = _pick_bm(m)

    def body(x_ref, o_ref):
        o_ref[...] = 1.0 / jnp.maximum(jnp.sum(x_ref[...], axis=0), lo)

    return pl.pallas_call(
        body, grid=(m // bm,),
        in_specs=[pl.BlockSpec((k, bm), lambda i: (0, i))],
        out_specs=pl.BlockSpec((bm,), lambda i: (i,)),
        out_shape=jax.ShapeDtypeStruct((m,), jnp.float32),
    )(x)


def _tanh_bias2(zh, b):
    _, n, d = zh.shape

    def body(z_ref, b_ref, o_ref):
        o_ref[...] = jnp.tanh(z_ref[0] + z_ref[1] + b_ref[...])

    return pl.pallas_call(
        body, grid=(n // _NROW,),
        in_specs=[
            pl.BlockSpec((2, _NROW, d), lambda i: (0, i, 0)),
            pl.BlockSpec((1, d), lambda i: (0, 0)),
        ],
        out_specs=pl.BlockSpec((_NROW, d), lambda i: (i, 0)),
        out_shape=jax.ShapeDtypeStruct((n, d), jnp.float32),
    )(zh, b)


def _dhw_chunk(msum, des, ded, asrc_c, adst_c):
    m, d = msum.shape
    bm = _pick_bm(m)

    def body(m_ref, s_ref, t_ref, as_ref, ad_ref, o_ref):
        o_ref[...] = (m_ref[...] + s_ref[...] * as_ref[...]
                      + t_ref[...] * ad_ref[...])

    return pl.pallas_call(
        body, grid=(m // bm,),
        in_specs=[
            pl.BlockSpec((bm, d), lambda i: (i, 0)),
            pl.BlockSpec((bm, 1), lambda i: (i, 0)),
            pl.BlockSpec((bm, 1), lambda i: (i, 0)),
            pl.BlockSpec((1, d), lambda i: (0, 0)),
            pl.BlockSpec((1, d), lambda i: (0, 0)),
        ],
        out_specs=pl.BlockSpec((bm, d), lambda i: (i, 0)),
        out_shape=jax.ShapeDtypeStruct((m, d), jnp.float32),
    )(msum, des, ded, asrc_c, adst_c)


def _mm_tanh_bwd(dhw, wt, h):
    n, dout = dhw.shape
    din = wt.shape[1]

    def body(d_ref, w_ref, h_ref, o_ref):
        dh = jnp.dot(d_ref[...], w_ref[...], preferred_element_type=jnp.float32)
        hh = h_ref[...]
        o_ref[...] = dh * (1.0 - hh * hh)

    return pl.pallas_call(
        body, grid=(n // _NROW,),
        in_specs=[
            pl.BlockSpec((_NROW, dout), lambda i: (i, 0)),
            pl.BlockSpec((dout, din), lambda i: (0, 0)),
            pl.BlockSpec((_NROW, din), lambda i: (i, 0)),
        ],
        out_specs=pl.BlockSpec((_NROW, din), lambda i: (i, 0)),
        out_shape=jax.ShapeDtypeStruct((n, din), jnp.float32),
    )(dhw, wt, h)


def _mm_plain(dhw, wt):
    n, dout = dhw.shape
    din = wt.shape[1]

    def body(d_ref, w_ref, o_ref):
        o_ref[...] = jnp.dot(d_ref[...], w_ref[...],
                             preferred_element_type=jnp.float32)

    return pl.pallas_call(
        body, grid=(n // _NROW,),
        in_specs=[
            pl.BlockSpec((_NROW, dout), lambda i: (i, 0)),
            pl.BlockSpec((dout, din), lambda i: (0, 0)),
        ],
        out_specs=pl.BlockSpec((_NROW, din), lambda i: (i, 0)),
        out_shape=jax.ShapeDtypeStruct((n, din), jnp.float32),
    )(dhw, wt)


def _gout3_build(g3n, hdim):
    n = g3n.shape[0]

    def body(g_ref, o_ref):
        colmask = (lax.broadcasted_iota(jnp.int32, (_NROW, 16), 1)
                   < hdim).astype(jnp.float32)
        o_ref[...] = g_ref[...] * colmask

    return pl.pallas_call(
        body, grid=(n // _NROW,),
        in_specs=[pl.BlockSpec((_NROW, 1), lambda i: (i, 0))],
        out_specs=pl.BlockSpec((_NROW, 16), lambda i: (i, 0)),
        out_shape=jax.ShapeDtypeStruct((n, 16), jnp.float32),
    )(g3n)


def _gat_forward(h, W, a_src, a_dst, src4, dst4, src_sc, dst_sc, emask,
                 n, nt_pad, need_out):
    dout = W.shape[1]
    a2 = jnp.stack([a_src, a_dst], axis=1)
    hw, esd = _mm_attn(h, W, a2)
    es = jnp.ascontiguousarray(esd[:, 0])
    ed = jnp.ascontiguousarray(esd[:, 1])
    t1 = _gather_scalar(es, src4)
    t2 = _gather_scalar(ed, dst4)
    e = _leaky(t1, t2)
    gm = _gmax(e)
    ex = _exp_shift_mask(e, gm, emask)
    den = _scatter_scalar(ex.reshape(NW, EC, CE), dst_sc, nt_pad)
    invd = _inv_clip(den, 1e-38)
    invd_e = _gather_scalar(invd, dst4)
    alpha = _mul(ex, invd_e)

    zh = None
    if need_out:
        nchunk = dout // 16
        halves = []
        for c in range(nchunk):
            hw_c = jnp.ascontiguousarray(hw[:, 16 * c:16 * (c + 1)])
            rows = _gather_rows(hw_c, src4)
            v = _scale_rows(rows, alpha[:, None])
            halves.append(_scatter_rows(v.reshape(NW, EC, CE, 16), dst_sc,
                                        nt_pad))
        zh = jnp.concatenate(halves, axis=-1)
    return zh, (hw, alpha, e)


def _gat_backward(gout, W, a_src, a_dst, saved, src4, dst4, src_sc, dst_sc,
                  n, nt_pad):
    hw, alpha, e = saved
    dout = W.shape[1]
    nchunk = dout // 16
    alpha_col = alpha[:, None]

    q_parts, msums = [], []
    for c in range(nchunk):
        g_c = jnp.ascontiguousarray(gout[:, 16 * c:16 * (c + 1)])
        hw_c = jnp.ascontiguousarray(hw[:, 16 * c:16 * (c + 1)])
        g_rows = _gather_rows(g_c, dst4)
        h_rows = _gather_rows(hw_c, src4)
        q_parts.append(_dot_rows(g_rows, h_rows))
        v = _scale_rows(g_rows, alpha_col)
        mh = _scatter_rows(v.reshape(NW, EC, CE, 16), src_sc, nt_pad)
        msums.append(_sum_leading(
            mh.reshape(2, nt_pad * 16)).reshape(nt_pad, 16))
    q = _sum_leading(jnp.stack(q_parts)) if nchunk > 1 else q_parts[0]
    r = _mul(alpha, q)
    sh = _scatter_scalar(r.reshape(NW, EC, CE), dst_sc, nt_pad)
    s = _sum_leading(sh)
    s_e = _gather_scalar(s, dst4)
    gl = _gl_fuse(r, alpha, s_e, e)
    desh = _scatter_scalar(gl.reshape(NW, EC, CE), src_sc, nt_pad)
    dedh = _scatter_scalar(gl.reshape(NW, EC, CE), dst_sc, nt_pad)
    des = _sum_leading(desh)[:, None]
    ded = _sum_leading(dedh)[:, None]

    dhw_chunks = []
    for c in range(nchunk):
        asrc_c = a_src[None, 16 * c:16 * (c + 1)]
        adst_c = a_dst[None, 16 * c:16 * (c + 1)]
        dhw_chunks.append(
            _dhw_chunk(msums[c], des, ded, asrc_c, adst_c)[:n])
    return jnp.concatenate(dhw_chunks, axis=-1)


def kernel(t, x, edge_index, graph_ids, W1, a1_src, a1_dst, b1,
           W2, a2_src, a2_dst, b2, W3, a3_src, a3_dst, b3):
    n = x.shape[0]
    e_real = edge_index.shape[1]
    hid = W1.shape[1]
    hdim = W3.shape[1]
    ngraph = 16
    nt_pad = ((n + 127) // 128) * 128

    src = edge_index[0]
    dst = edge_index[1]
    pad_e = E_PAD - e_real
    src_p = jnp.pad(src, (0, pad_e))
    dst_p = jnp.pad(dst, (0, pad_e))
    src4 = src_p.reshape(NW, EC, KB, 128)
    dst4 = dst_p.reshape(NW, EC, KB, 128)
    emask = jnp.pad(jnp.ones((e_real,), jnp.float32), (0, pad_e))

    gid_p = jnp.pad(graph_ids, (0, N_PAD - n))
    gid4 = gid_p.reshape(NW, 1, KN, 128)
    nmask = jnp.pad(jnp.ones((n,), jnp.float32), (0, N_PAD - n))

    W3p = jnp.pad(W3, ((0, 0), (0, 16 - hdim)))
    a3s = jnp.pad(a3_src, (0, 16 - hdim))
    a3d = jnp.pad(a3_dst, (0, 16 - hdim))

    zh1, sv1 = _gat_forward(x, W1, a1_src, a1_dst, src4, dst4, src4, dst4,
                            emask, n, nt_pad, True)
    h1 = _tanh_bias2(zh1[:, :n, :], b1[None, :])
    zh2, sv2 = _gat_forward(h1, W2, a2_src, a2_dst, src4, dst4, src4, dst4,
                            emask, n, nt_pad, True)
    h2 = _tanh_bias2(zh2[:, :n, :], b2[None, :])
    _, sv3 = _gat_forward(h2, W3p, a3s, a3d, src4, dst4, src4, dst4,
                          emask, n, nt_pad, False)

    cnt_h = _scatter_scalar(nmask.reshape(NW, 1, CN), gid4, 128)
    invc = _inv_clip(cnt_h, 1.0)
    g3n = _gather_scalar(invc, gid4)[:n]
    gout3 = _gout3_build(g3n[:, None], hdim)

    dhw3 = _gat_backward(gout3, W3p, a3s, a3d, sv3, src4, dst4, src4, dst4,
                         n, nt_pad)
    dz2 = _mm_tanh_bwd(dhw3, W3p.T, h2)
    dhw2 = _gat_backward(dz2, W2, a2_src, a2_dst, sv2, src4, dst4, src4,
                         dst4, n, nt_pad)
    dz1 = _mm_tanh_bwd(dhw2, W2.T, h1)
    dhw1 = _gat_backward(dz1, W1, a1_src, a1_dst, sv1, src4, dst4, src4,
                         dst4, n, nt_pad)
    dx = _mm_plain(dhw1, W1.T)
    return dx


# speedup vs baseline: 9.0270x; 9.0270x over previous
import functools

import jax
import jax.numpy as jnp
from jax import lax
from jax.experimental import pallas as pl
from jax.experimental.pallas import tpu as pltpu
from jax.experimental.pallas import tpu_sc as plsc

NC = 2
NS = 16
NW = NC * NS
LANES = 16

CE = 6400
KB = CE // 128
EC = 8
PER_W = EC * CE
E_PAD = NW * PER_W

CN = 3200
KN = CN // 128
N_PAD = NW * CN


def _sc_mesh():
    return plsc.VectorSubcoreMesh(core_axis_name="c", subcore_axis_name="s",
                                  num_cores=NC)


@functools.lru_cache(maxsize=None)
def _make_gather_rows(nt, d, nchunks, ce):
    @functools.partial(
        pl.kernel, mesh=_sc_mesh(),
        compiler_params=pltpu.CompilerParams(use_tc_tiling_on_sc=False),
        out_type=jax.ShapeDtypeStruct((NW * nchunks * ce, d), jnp.float32),
        scratch_types=[
            pltpu.VMEM((ce,), jnp.int32),
            pltpu.VMEM((ce, d), jnp.float32),
            pltpu.SemaphoreType.DMA,
        ],
    )
    def k(table_hbm, idx_hbm, out_hbm, idx_v, rows_v, sem):
        w = lax.axis_index("s") * NC + lax.axis_index("c")

        def chunk(j, carry):
            base = (w * nchunks + j) * ce
            pltpu.sync_copy(idx_hbm.at[pl.ds(base, ce)], idx_v)
            pltpu.async_copy(table_hbm.at[idx_v], rows_v, sem).wait()
            pltpu.sync_copy(rows_v, out_hbm.at[pl.ds(base, ce)])
            return carry

        lax.fori_loop(0, nchunks, chunk, 0)

    return k


@functools.lru_cache(maxsize=None)
def _make_gather_scalar(nt, nchunks, ce):
    @functools.partial(
        pl.kernel, mesh=_sc_mesh(),
        compiler_params=pltpu.CompilerParams(use_tc_tiling_on_sc=False),
        out_type=jax.ShapeDtypeStruct((NW * nchunks * ce,), jnp.float32),
        scratch_types=[
            pltpu.VMEM((ce,), jnp.int32),
            pltpu.VMEM((ce,), jnp.float32),
            pltpu.SemaphoreType.DMA,
        ],
    )
    def k(table_hbm, idx_hbm, out_hbm, idx_v, out_v, sem):
        w = lax.axis_index("s") * NC + lax.axis_index("c")

        def chunk(j, carry):
            base = (w * nchunks + j) * ce
            pltpu.sync_copy(idx_hbm.at[pl.ds(base, ce)], idx_v)
            pltpu.async_copy(table_hbm.at[idx_v], out_v, sem).wait()
            pltpu.sync_copy(out_v, out_hbm.at[pl.ds(base, ce)])
            return carry

        lax.fori_loop(0, nchunks, chunk, 0)

    return k


@functools.lru_cache(maxsize=None)
def _make_edge_att(nt_pad, nchunks, ce):
    @functools.partial(
        pl.kernel, mesh=_sc_mesh(),
        compiler_params=pltpu.CompilerParams(use_tc_tiling_on_sc=False),
        out_type=[jax.ShapeDtypeStruct((NW * nchunks * ce,), jnp.float32),
                  jax.ShapeDtypeStruct((NW * 16,), jnp.float32)],
        scratch_types=[
            pltpu.VMEM((ce,), jnp.int32),
            pltpu.VMEM((ce,), jnp.int32),
            pltpu.VMEM((ce,), jnp.float32),
            pltpu.VMEM((ce,), jnp.float32),
            pltpu.VMEM((16,), jnp.float32),
            pltpu.SemaphoreType.DMA,
        ],
    )
    def k(es_hbm, ed_hbm, src_hbm, dst_hbm, e_hbm, pmax_hbm,
          i1, i2, t1, t2, mxv, sem):
        w = lax.axis_index("s") * NC + lax.axis_index("c")

        def chunk(j, mx):
            base = (w * nchunks + j) * ce
            pltpu.sync_copy(src_hbm.at[pl.ds(base, ce)], i1)
            pltpu.sync_copy(dst_hbm.at[pl.ds(base, ce)], i2)
            h1 = pltpu.async_copy(es_hbm.at[i1], t1, sem)
            h2 = pltpu.async_copy(ed_hbm.at[i2], t2, sem)
            h1.wait()
            h2.wait()

            def step(i, mx2):
                sl = pl.ds(i * 16, 16)
                ssum = t1[sl] + t2[sl]
                ev = jnp.where(ssum >= 0, ssum, 0.2 * ssum)
                t1[sl] = ev
                return jnp.maximum(mx2, ev)

            mx = lax.fori_loop(0, ce // 16, step, mx)
            pltpu.sync_copy(t1, e_hbm.at[pl.ds(base, ce)])
            return mx

        mx0 = jnp.full((16,), -jnp.inf, jnp.float32)
        mx = lax.fori_loop(0, nchunks, chunk, mx0)
        mxv[...] = mx
        pltpu.sync_copy(mxv, pmax_hbm.at[pl.ds(w * 16, 16)])

    return k


@functools.lru_cache(maxsize=None)
def _make_exp_denom(nt_pad, nchunks, ce):
    kb = ce // 128
    stripe = nt_pad // NS

    @functools.partial(
        pl.kernel, mesh=_sc_mesh(),
        compiler_params=pltpu.CompilerParams(use_tc_tiling_on_sc=False),
        out_type=[jax.ShapeDtypeStruct((NW * nchunks * ce,), jnp.float32),
                  jax.ShapeDtypeStruct((NC * nt_pad,), jnp.float32)],
        scratch_types=[
            pltpu.VMEM_SHARED((nt_pad,), jnp.float32),
            pltpu.VMEM((kb, 128), jnp.int32),
            pltpu.VMEM((ce,), jnp.float32),
            pltpu.VMEM((ce,), jnp.float32),
            pltpu.VMEM((16,), jnp.float32),
            pltpu.SemaphoreType.DMA,
        ],
    )
    def k(e_hbm, gm_hbm, m_hbm, dst2_hbm, ex_hbm, den_hbm,
          acc, idx_v, ev, mv, gmv, sem):
        c = lax.axis_index("c")
        s = lax.axis_index("s")
        w = s * NC + c

        zero = jnp.zeros((16,), jnp.float32)

        def zrow(i, carry):
            ev[pl.ds(i * 16, 16)] = zero
            return carry

        lax.fori_loop(0, stripe // 16, zrow, 0)
        pltpu.sync_copy(ev.at[pl.ds(0, stripe)],
                        acc.at[pl.ds(s * stripe, stripe)])
        pltpu.sync_copy(gm_hbm, gmv)
        plsc.subcore_barrier()
        g = gmv[...]

        def chunk(j, carry):
            base = (w * nchunks + j) * ce
            pltpu.sync_copy(e_hbm.at[pl.ds(base, ce)], ev)
            pltpu.sync_copy(m_hbm.at[pl.ds(base, ce)], mv)
            pltpu.sync_copy(dst2_hbm.at[w, j], idx_v)

            def step(i, c2):
                sl = pl.ds(i * 16, 16)
                ev[sl] = jnp.exp(ev[sl] - g) * mv[sl]
                return c2

            lax.fori_loop(0, ce // 16, step, 0)
            pltpu.sync_copy(ev, ex_hbm.at[pl.ds(base, ce)])
            handles = []
            for b in range(kb):
                handles.append(pltpu.async_copy(
                    ev.at[pl.ds(b * 128, 128)],
                    acc.at[idx_v.at[b]], sem, add=True))
            for h in handles:
                h.wait()
            return carry

        lax.fori_loop(0, nchunks, chunk, 0)
        plsc.subcore_barrier()
        pltpu.sync_copy(acc.at[pl.ds(s * stripe, stripe)],
                        den_hbm.at[pl.ds(c * nt_pad + s * stripe, stripe)])

    return k


@functools.lru_cache(maxsize=None)
def _make_alpha(nt_pad, nchunks, ce):
    @functools.partial(
        pl.kernel, mesh=_sc_mesh(),
        compiler_params=pltpu.CompilerParams(use_tc_tiling_on_sc=False),
        out_type=jax.ShapeDtypeStruct((NW * nchunks * ce,), jnp.float32),
        scratch_types=[
            pltpu.VMEM((ce,), jnp.int32),
            pltpu.VMEM((ce,), jnp.float32),
            pltpu.VMEM((ce,), jnp.float32),
            pltpu.SemaphoreType.DMA,
        ],
    )
    def k(invd_hbm, dst_hbm, ex_hbm, out_hbm, i1, t1, t2, sem):
        w = lax.axis_index("s") * NC + lax.axis_index("c")

        def chunk(j, carry):
            base = (w * nchunks + j) * ce
            pltpu.sync_copy(dst_hbm.at[pl.ds(base, ce)], i1)
            h1 = pltpu.async_copy(invd_hbm.at[i1], t1, sem)
            pltpu.sync_copy(ex_hbm.at[pl.ds(base, ce)], t2)
            h1.wait()

            def step(i, c2):
                sl = pl.ds(i * 16, 16)
                t1[sl] = t1[sl] * t2[sl]
                return c2

            lax.fori_loop(0, ce // 16, step, 0)
            pltpu.sync_copy(t1, out_hbm.at[pl.ds(base, ce)])
            return carry

        lax.fori_loop(0, nchunks, chunk, 0)

    return k


@functools.lru_cache(maxsize=None)
def _make_gl(nt_pad, nchunks, ce):
    kb = ce // 128
    stripe = nt_pad // NS

    @functools.partial(
        pl.kernel, mesh=_sc_mesh(),
        compiler_params=pltpu.CompilerParams(use_tc_tiling_on_sc=False),
        out_type=[jax.ShapeDtypeStruct((NC * nt_pad,), jnp.float32),
                  jax.ShapeDtypeStruct((NC * nt_pad,), jnp.float32)],
        scratch_types=[
            pltpu.VMEM_SHARED((nt_pad,), jnp.float32),
            pltpu.VMEM_SHARED((nt_pad,), jnp.float32),
            pltpu.VMEM((ce,), jnp.int32),
            pltpu.VMEM((ce,), jnp.int32),
            pltpu.VMEM((ce,), jnp.float32),
            pltpu.VMEM((ce,), jnp.float32),
            pltpu.VMEM((ce,), jnp.float32),
            pltpu.VMEM((ce,), jnp.float32),
            pltpu.SemaphoreType.DMA,
        ],
    )
    def k(s_hbm, dst_hbm, r_hbm, a_hbm, e_hbm, src_hbm,
          des_hbm, ded_hbm, acc_s, acc_d, i1, i2, sv, rv, av, evv,
          sem):
        c = lax.axis_index("c")
        s = lax.axis_index("s")
        w = s * NC + c

        zero = jnp.zeros((16,), jnp.float32)

        def zrow(i, carry):
            rv[pl.ds(i * 16, 16)] = zero
            return carry

        lax.fori_loop(0, stripe // 16, zrow, 0)
        pltpu.sync_copy(rv.at[pl.ds(0, stripe)],
                        acc_s.at[pl.ds(s * stripe, stripe)])
        pltpu.sync_copy(rv.at[pl.ds(0, stripe)],
                        acc_d.at[pl.ds(s * stripe, stripe)])
        plsc.subcore_barrier()

        def chunk(j, carry):
            base = (w * nchunks + j) * ce
            pltpu.sync_copy(dst_hbm.at[pl.ds(base, ce)], i1)
            h1 = pltpu.async_copy(s_hbm.at[i1], sv, sem)
            pltpu.sync_copy(r_hbm.at[pl.ds(base, ce)], rv)
            pltpu.sync_copy(a_hbm.at[pl.ds(base, ce)], av)
            pltpu.sync_copy(e_hbm.at[pl.ds(base, ce)], evv)
            pltpu.sync_copy(src_hbm.at[pl.ds(base, ce)], i2)
            h1.wait()

            def step(i, c2):
                sl = pl.ds(i * 16, 16)
                ge = rv[sl] - av[sl] * sv[sl]
                rv[sl] = ge * jnp.where(evv[sl] >= 0, 1.0, 0.2)
                return c2

            lax.fori_loop(0, ce // 16, step, 0)
            h1 = pltpu.async_copy(rv, acc_s.at[i2], sem, add=True)
            h2 = pltpu.async_copy(rv, acc_d.at[i1], sem, add=True)
            h1.wait()
            h2.wait()
            return carry

        lax.fori_loop(0, nchunks, chunk, 0)
        plsc.subcore_barrier()
        pltpu.sync_copy(acc_s.at[pl.ds(s * stripe, stripe)],
                        des_hbm.at[pl.ds(c * nt_pad + s * stripe, stripe)])
        pltpu.sync_copy(acc_d.at[pl.ds(s * stripe, stripe)],
                        ded_hbm.at[pl.ds(c * nt_pad + s * stripe, stripe)])

    return k


@functools.lru_cache(maxsize=None)
def _make_scatter_rows(nt_pad, d, nchunks, ce):
    kb = ce // 128
    stripe = nt_pad // NS

    @functools.partial(
        pl.kernel, mesh=_sc_mesh(),
        compiler_params=pltpu.CompilerParams(use_tc_tiling_on_sc=False),
        out_type=jax.ShapeDtypeStruct((NC, nt_pad, d), jnp.float32),
        scratch_types=[
            pltpu.VMEM_SHARED((nt_pad, d), jnp.float32),
            pltpu.VMEM((kb, 128), jnp.int32),
            pltpu.VMEM((ce, d), jnp.float32),
            pltpu.SemaphoreType.DMA,
        ],
    )
    def k(vals_hbm, idx_hbm, out_hbm, acc, idx_v, vals_v, sem):
        c = lax.axis_index("c")
        s = lax.axis_index("s")
        w = s * NC + c

        zero = jnp.zeros((16,), jnp.float32)

        def zrow(i, carry):
            vals_v[i] = zero
            return carry

        lax.fori_loop(0, ce, zrow, 0)
        for off in range(0, stripe, ce):
            sz = min(ce, stripe - off)
            pltpu.sync_copy(vals_v.at[pl.ds(0, sz)],
                            acc.at[pl.ds(s * stripe + off, sz)])
        plsc.subcore_barrier()

        def chunk(j, carry):
            pltpu.sync_copy(idx_hbm.at[w, j], idx_v)
            pltpu.sync_copy(vals_hbm.at[w, j], vals_v)
            handles = []
            for b in range(kb):
                handles.append(pltpu.async_copy(
                    vals_v.at[pl.ds(b * 128, 128)],
                    acc.at[idx_v.at[b]], sem, add=True))
            for h in handles:
                h.wait()
            return carry

        lax.fori_loop(0, nchunks, chunk, 0)
        plsc.subcore_barrier()
        pltpu.sync_copy(acc.at[pl.ds(s * stripe, stripe)],
                        out_hbm.at[c, pl.ds(s * stripe, stripe)])

    return k


@functools.lru_cache(maxsize=None)
def _make_scatter_scalar(nt_pad, nchunks, ce):
    kb = ce // 128
    small = nt_pad < NS * 128
    stripe = nt_pad if small else nt_pad // NS

    @functools.partial(
        pl.kernel, mesh=_sc_mesh(),
        compiler_params=pltpu.CompilerParams(use_tc_tiling_on_sc=False),
        out_type=jax.ShapeDtypeStruct((NC * nt_pad,), jnp.float32),
        scratch_types=[
            pltpu.VMEM_SHARED((nt_pad,), jnp.float32),
            pltpu.VMEM((kb, 128), jnp.int32),
            pltpu.VMEM((ce,), jnp.float32),
            pltpu.SemaphoreType.DMA,
        ],
    )
    def k(vals_hbm, idx_hbm, out_hbm, acc, idx_v, vals_v, sem):
        c = lax.axis_index("c")
        s = lax.axis_index("s")
        w = s * NC + c

        zero = jnp.zeros((16,), jnp.float32)

        def zrow(i, carry):
            vals_v[pl.ds(i * 16, 16)] = zero
            return carry

        lax.fori_loop(0, stripe // 16, zrow, 0)
        if small:
            @pl.when(s == 0)
            def _():
                pltpu.sync_copy(vals_v.at[pl.ds(0, stripe)], acc)
        else:
            pltpu.sync_copy(vals_v.at[pl.ds(0, stripe)],
                            acc.at[pl.ds(s * stripe, stripe)])
        plsc.subcore_barrier()

        def chunk(j, carry):
            pltpu.sync_copy(idx_hbm.at[w, j], idx_v)
            pltpu.sync_copy(vals_hbm.at[w, j], vals_v)
            handles = []
            for b in range(kb):
                handles.append(pltpu.async_copy(
                    vals_v.at[pl.ds(b * 128, 128)],
                    acc.at[idx_v.at[b]], sem, add=True))
            for h in handles:
                h.wait()
            return carry

        lax.fori_loop(0, nchunks, chunk, 0)
        plsc.subcore_barrier()
        if small:
            @pl.when(s == 0)
            def _():
                pltpu.sync_copy(acc, out_hbm.at[pl.ds(c * nt_pad, nt_pad)])
        else:
            pltpu.sync_copy(
                acc.at[pl.ds(s * stripe, stripe)],
                out_hbm.at[pl.ds(c * nt_pad + s * stripe, stripe)])

    return k


def _edge_ce(total, cap):
    per_w = total // NW
    ce = cap
    while per_w % ce or ce % 128:
        ce -= 128
    return per_w // ce, ce


def _gather_rows(table, idx_flat):
    d = table.shape[1]
    cap = 81920 // d // 128 * 128
    nchunks, ce = _edge_ce(idx_flat.shape[0], min(cap, 6400))
    return _make_gather_rows(table.shape[0], d, nchunks, ce)(table, idx_flat)


def _gather_scalar(table, idx_flat):
    nchunks, ce = _edge_ce(idx_flat.shape[0], 6400)
    return _make_gather_scalar(table.shape[0], nchunks, ce)(table, idx_flat)


def _edge_att(es, ed, srcp, dstp):
    nchunks, ce = _edge_ce(srcp.shape[0], 6400)
    return _make_edge_att(es.shape[0], nchunks, ce)(es, ed, srcp, dstp)


def _exp_denom(e, gm16, mask, dstp, nt_pad):
    nchunks, ce = _edge_ce(dstp.shape[0], 6400)
    dst2 = dstp.reshape(NW, nchunks, ce // 128, 128)
    ex, den = _make_exp_denom(nt_pad, nchunks, ce)(e, gm16, mask, dst2)
    return ex, den.reshape(NC, nt_pad)


def _alpha_e(invd, dstp, ex):
    nchunks, ce = _edge_ce(dstp.shape[0], 6400)
    return _make_alpha(invd.shape[0], nchunks, ce)(invd, dstp, ex)


def _gl_seg(s, dstp, r, alpha, e, srcp, nt_pad):
    nchunks, ce = _edge_ce(dstp.shape[0], 6400)
    desf, dedf = _make_gl(nt_pad, nchunks, ce)(s, dstp, r, alpha, e, srcp)
    return desf.reshape(NC, nt_pad), dedf.reshape(NC, nt_pad)


def _scatter_rows(vals, idx_flat, nt_pad):
    d = vals.shape[1]
    nchunks, ce = _edge_ce(idx_flat.shape[0], 1280)
    vals3 = vals.reshape(NW, nchunks, ce, d)
    idx4 = idx_flat.reshape(NW, nchunks, ce // 128, 128)
    return _make_scatter_rows(nt_pad, d, nchunks, ce)(vals3, idx4)


def _scatter_scalar(vals, idx_flat, nt_pad):
    nchunks, ce = _edge_ce(idx_flat.shape[0], 6400)
    vals3 = vals.reshape(NW, nchunks, ce)
    idx4 = idx_flat.reshape(NW, nchunks, ce // 128, 128)
    flat = _make_scatter_scalar(nt_pad, nchunks, ce)(vals3, idx4)
    return flat.reshape(NC, nt_pad)


_NROW = 1000


def _mm_attn(h, W, a2):
    n, din = h.shape
    dout = W.shape[1]

    def body(h_ref, w_ref, a_ref, hw_ref, esd_ref):
        hw = jnp.dot(h_ref[...], w_ref[...], preferred_element_type=jnp.float32, precision=lax.Precision.HIGHEST)
        hw_ref[...] = hw
        esd_ref[...] = jnp.dot(hw, a_ref[...], preferred_element_type=jnp.float32, precision=lax.Precision.HIGHEST)

    return pl.pallas_call(
        body,
        grid=(n // _NROW,),
        in_specs=[
            pl.BlockSpec((_NROW, din), lambda i: (i, 0)),
            pl.BlockSpec((din, dout), lambda i: (0, 0)),
            pl.BlockSpec((dout, 2), lambda i: (0, 0)),
        ],
        out_specs=[
            pl.BlockSpec((_NROW, dout), lambda i: (i, 0)),
            pl.BlockSpec((_NROW, 2), lambda i: (i, 0)),
        ],
        out_shape=[
            jax.ShapeDtypeStruct((n, dout), jnp.float32),
            jax.ShapeDtypeStruct((n, 2), jnp.float32),
        ],
    )(h, W, a2)


def _ew_grid(m, bm=16384):
    if m < bm:
        return 1, m
    assert m % bm == 0, (m, bm)
    return m // bm, bm


def _leaky(t1, t2):
    g, bm = _ew_grid(t1.shape[0])

    def body(a_ref, b_ref, o_ref):
        s = a_ref[...] + b_ref[...]
        o_ref[...] = jnp.where(s >= 0, s, 0.2 * s)

    return pl.pallas_call(
        body, grid=(g,),
        in_specs=[pl.BlockSpec((bm,), lambda i: (i,))] * 2,
        out_specs=pl.BlockSpec((bm,), lambda i: (i,)),
        out_shape=jax.ShapeDtypeStruct(t1.shape, jnp.float32),
    )(t1, t2)


def _gmax(e):
    g, bm = _ew_grid(e.shape[0])

    def body(e_ref, o_ref):
        i = pl.program_id(0)

        @pl.when(i == 0)
        def _():
            o_ref[...] = jnp.full((1, 1), -jnp.inf, jnp.float32)

        o_ref[...] = jnp.maximum(o_ref[...], jnp.max(e_ref[...]))

    return pl.pallas_call(
        body, grid=(g,),
        in_specs=[pl.BlockSpec((bm,), lambda i: (i,))],
        out_specs=pl.BlockSpec((1, 1), lambda i: (0, 0)),
        out_shape=jax.ShapeDtypeStruct((1, 1), jnp.float32),
    )(e)


def _exp_shift_mask(e, gm, mask):
    g, bm = _ew_grid(e.shape[0])

    def body(e_ref, g_ref, m_ref, o_ref):
        o_ref[...] = jnp.exp(e_ref[...] - g_ref[0, 0]) * m_ref[...]

    return pl.pallas_call(
        body, grid=(g,),
        in_specs=[
            pl.BlockSpec((bm,), lambda i: (i,)),
            pl.BlockSpec((1, 1), lambda i: (0, 0)),
            pl.BlockSpec((bm,), lambda i: (i,)),
        ],
        out_specs=pl.BlockSpec((bm,), lambda i: (i,)),
        out_shape=jax.ShapeDtypeStruct(e.shape, jnp.float32),
    )(e, gm, mask)


def _mul(a, b):
    g, bm = _ew_grid(a.shape[0])

    def body(a_ref, b_ref, o_ref):
        o_ref[...] = a_ref[...] * b_ref[...]

    return pl.pallas_call(
        body, grid=(g,),
        in_specs=[pl.BlockSpec((bm,), lambda i: (i,))] * 2,
        out_specs=pl.BlockSpec((bm,), lambda i: (i,)),
        out_shape=jax.ShapeDtypeStruct(a.shape, jnp.float32),
    )(a, b)


def _gl_fuse(r, alpha, s_e, e):
    g, bm = _ew_grid(r.shape[0])

    def body(r_ref, a_ref, s_ref, e_ref, o_ref):
        ge = r_ref[...] - a_ref[...] * s_ref[...]
        o_ref[...] = ge * jnp.where(e_ref[...] >= 0, 1.0, 0.2)

    return pl.pallas_call(
        body, grid=(g,),
        in_specs=[pl.BlockSpec((bm,), lambda i: (i,))] * 4,
        out_specs=pl.BlockSpec((bm,), lambda i: (i,)),
        out_shape=jax.ShapeDtypeStruct(r.shape, jnp.float32),
    )(r, alpha, s_e, e)


def _scale_rows(rows, s):
    m, d = rows.shape
    bm = 16384
    assert m % bm == 0

    def body(r_ref, s_ref, o_ref):
        o_ref[...] = r_ref[...] * s_ref[...]

    return pl.pallas_call(
        body, grid=(m // bm,),
        in_specs=[
            pl.BlockSpec((bm, d), lambda i: (i, 0)),
            pl.BlockSpec((bm, 1), lambda i: (i, 0)),
        ],
        out_specs=pl.BlockSpec((bm, d), lambda i: (i, 0)),
        out_shape=jax.ShapeDtypeStruct((m, d), jnp.float32),
    )(rows, s)


def _dot_rows(a, b):
    m, d = a.shape
    bm = 16384
    assert m % bm == 0

    def body(a_ref, b_ref, o_ref):
        o_ref[...] = jnp.sum(a_ref[...] * b_ref[...], axis=-1)

    return pl.pallas_call(
        body, grid=(m // bm,),
        in_specs=[pl.BlockSpec((bm, d), lambda i: (i, 0))] * 2,
        out_specs=pl.BlockSpec((bm,), lambda i: (i,)),
        out_shape=jax.ShapeDtypeStruct((m,), jnp.float32),
    )(a, b)


def _scale_split(rows, s):
    m, d = rows.shape
    nch = d // 16
    bm = 4096

    def body(r_ref, s_ref, *o_refs):
        v = r_ref[...] * s_ref[...]
        for c in range(nch):
            o_refs[c][...] = v[:, 16 * c:16 * (c + 1)]

    return pl.pallas_call(
        body, grid=(m // bm,),
        in_specs=[
            pl.BlockSpec((bm, d), lambda i: (i, 0)),
            pl.BlockSpec((bm, 1), lambda i: (i, 0)),
        ],
        out_specs=[pl.BlockSpec((bm, 16), lambda i: (i, 0))] * nch,
        out_shape=[jax.ShapeDtypeStruct((m, 16), jnp.float32)] * nch,
    )(rows, s)


def _dot_scale(a, b, al):
    m, d = a.shape
    bm = 4096

    def body(a_ref, b_ref, s_ref, o_ref):
        o_ref[...] = jnp.sum(a_ref[...] * b_ref[...], axis=-1) * s_ref[:, 0]

    return pl.pallas_call(
        body, grid=(m // bm,),
        in_specs=[
            pl.BlockSpec((bm, d), lambda i: (i, 0)),
            pl.BlockSpec((bm, d), lambda i: (i, 0)),
            pl.BlockSpec((bm, 1), lambda i: (i, 0)),
        ],
        out_specs=pl.BlockSpec((bm,), lambda i: (i,)),
        out_shape=jax.ShapeDtypeStruct((m,), jnp.float32),
    )(a, b, al)


def _pick_bm(m):
    for bm in (16384, 8192, 4096, 2048, 1024, 128):
        if m % bm == 0:
            return bm
    raise ValueError(m)


def _sum_leading(x):
    k, m = x.shape
    bm = _pick_bm(m)

    def body(x_ref, o_ref):
        o_ref[...] = jnp.sum(x_ref[...], axis=0)

    return pl.pallas_call(
        body, grid=(m // bm,),
        in_specs=[pl.BlockSpec((k, bm), lambda i: (0, i))],
        out_specs=pl.BlockSpec((bm,), lambda i: (i,)),
        out_shape=jax.ShapeDtypeStruct((m,), jnp.float32),
    )(x)


def _inv_clip(x, lo):
    k, m = x.shape
    bm = _pick_bm(m)

    def body(x_ref, o_ref):
        o_ref[...] = 1.0 / jnp.maximum(jnp.sum(x_ref[...], axis=0), lo)

    return pl.pallas_call(
        body, grid=(m // bm,),
        in_specs=[pl.BlockSpec((k, bm), lambda i: (0, i))],
        out_specs=pl.BlockSpec((bm,), lambda i: (i,)),
        out_shape=jax.ShapeDtypeStruct((m,), jnp.float32),
    )(x)


def _tanh_bias2(zh, b):
    _, n, d = zh.shape

    def body(z_ref, b_ref, o_ref):
        o_ref[...] = jnp.tanh(z_ref[0] + z_ref[1] + b_ref[...])

    return pl.pallas_call(
        body, grid=(n // _NROW,),
        in_specs=[
            pl.BlockSpec((2, _NROW, d), lambda i: (0, i, 0)),
            pl.BlockSpec((1, d), lambda i: (0, 0)),
        ],
        out_specs=pl.BlockSpec((_NROW, d), lambda i: (i, 0)),
        out_shape=jax.ShapeDtypeStruct((n, d), jnp.float32),
    )(zh, b)


def _dhw_chunk(msum, des, ded, asrc_c, adst_c):
    m, d = msum.shape
    bm = _pick_bm(m)

    def body(m_ref, s_ref, t_ref, as_ref, ad_ref, o_ref):
        o_ref[...] = (m_ref[...] + s_ref[...] * as_ref[...]
                      + t_ref[...] * ad_ref[...])

    return pl.pallas_call(
        body, grid=(m // bm,),
        in_specs=[
            pl.BlockSpec((bm, d), lambda i: (i, 0)),
            pl.BlockSpec((bm, 1), lambda i: (i, 0)),
            pl.BlockSpec((bm, 1), lambda i: (i, 0)),
            pl.BlockSpec((1, d), lambda i: (0, 0)),
            pl.BlockSpec((1, d), lambda i: (0, 0)),
        ],
        out_specs=pl.BlockSpec((bm, d), lambda i: (i, 0)),
        out_shape=jax.ShapeDtypeStruct((m, d), jnp.float32),
    )(msum, des, ded, asrc_c, adst_c)


def _mm_tanh_bwd(dhw, wt, h):
    n, dout = dhw.shape
    din = wt.shape[1]

    def body(d_ref, w_ref, h_ref, o_ref):
        dh = jnp.dot(d_ref[...], w_ref[...], preferred_element_type=jnp.float32, precision=lax.Precision.HIGHEST)
        hh = h_ref[...]
        o_ref[...] = dh * (1.0 - hh * hh)

    return pl.pallas_call(
        body, grid=(n // _NROW,),
        in_specs=[
            pl.BlockSpec((_NROW, dout), lambda i: (i, 0)),
            pl.BlockSpec((dout, din), lambda i: (0, 0)),
            pl.BlockSpec((_NROW, din), lambda i: (i, 0)),
        ],
        out_specs=pl.BlockSpec((_NROW, din), lambda i: (i, 0)),
        out_shape=jax.ShapeDtypeStruct((n, din), jnp.float32),
    )(dhw, wt, h)


def _mm_plain(dhw, wt):
    n, dout = dhw.shape
    din = wt.shape[1]

    def body(d_ref, w_ref, o_ref):
        o_ref[...] = jnp.dot(d_ref[...], w_ref[...],
                             preferred_element_type=jnp.float32, precision=lax.Precision.HIGHEST)

    return pl.pallas_call(
        body, grid=(n // _NROW,),
        in_specs=[
            pl.BlockSpec((_NROW, dout), lambda i: (i, 0)),
            pl.BlockSpec((dout, din), lambda i: (0, 0)),
        ],
        out_specs=pl.BlockSpec((_NROW, din), lambda i: (i, 0)),
        out_shape=jax.ShapeDtypeStruct((n, din), jnp.float32),
    )(dhw, wt)


def _gout3_build(g3n, hdim):
    n = g3n.shape[0]

    def body(g_ref, o_ref):
        colmask = (lax.broadcasted_iota(jnp.int32, (_NROW, 16), 1)
                   < hdim).astype(jnp.float32)
        o_ref[...] = g_ref[...] * colmask

    return pl.pallas_call(
        body, grid=(n // _NROW,),
        in_specs=[pl.BlockSpec((_NROW, 1), lambda i: (i, 0))],
        out_specs=pl.BlockSpec((_NROW, 16), lambda i: (i, 0)),
        out_shape=jax.ShapeDtypeStruct((n, 16), jnp.float32),
    )(g3n)


def _gat_forward(h, W, a_src, a_dst, srcp, dstp, emask,
                 n, nt_pad, need_out):
    dout = W.shape[1]
    a2 = jnp.stack([a_src, a_dst], axis=1)
    hw, esd = _mm_attn(h, W, a2)
    es = jnp.pad(esd[:, 0], (0, nt_pad - n))
    ed = jnp.pad(esd[:, 1], (0, nt_pad - n))
    e, pmax = _edge_att(es, ed, srcp, dstp)
    gm = _gmax(pmax)
    gm16 = jnp.broadcast_to(gm.reshape(1), (16,))
    ex, den = _exp_denom(e, gm16, emask, dstp, nt_pad)
    invd = _inv_clip(den, 1e-38)
    alpha = _alpha_e(invd, dstp, ex)

    zh = None
    if need_out:
        rows = _gather_rows(hw, srcp)
        vc = _scale_split(rows, alpha[:, None])
        halves = [_scatter_rows(v, dstp, nt_pad) for v in vc]
        zh = jnp.concatenate(halves, axis=-1)
    return zh, (hw, alpha, e)


def _gat_backward(gout, W, a_src, a_dst, saved, srcp, dstp, n, nt_pad):
    hw, alpha, e = saved
    dout = W.shape[1]
    nchunk = dout // 16
    alpha_col = alpha[:, None]

    g_rows = _gather_rows(gout, dstp)
    h_rows = _gather_rows(hw, srcp)
    r = _dot_scale(g_rows, h_rows, alpha_col)
    vc = _scale_split(g_rows, alpha_col)
    msums = []
    for c in range(nchunk):
        mh = _scatter_rows(vc[c], srcp, nt_pad)
        msums.append(_sum_leading(
            mh.reshape(2, nt_pad * 16)).reshape(nt_pad, 16))
    sh = _scatter_scalar(r, dstp, nt_pad)
    s = _sum_leading(sh)
    desh, dedh = _gl_seg(s, dstp, r, alpha, e, srcp, nt_pad)
    des = _sum_leading(desh)[:, None]
    ded = _sum_leading(dedh)[:, None]

    dhw_chunks = []
    for c in range(nchunk):
        asrc_c = a_src[None, 16 * c:16 * (c + 1)]
        adst_c = a_dst[None, 16 * c:16 * (c + 1)]
        dhw_chunks.append(
            _dhw_chunk(msums[c], des, ded, asrc_c, adst_c)[:n])
    return jnp.concatenate(dhw_chunks, axis=-1)


def kernel(t, x, edge_index, graph_ids, W1, a1_src, a1_dst, b1,
           W2, a2_src, a2_dst, b2, W3, a3_src, a3_dst, b3):
    n = x.shape[0]
    e_real = edge_index.shape[1]
    hid = W1.shape[1]
    hdim = W3.shape[1]
    ngraph = 16
    nt_pad = ((n + 2047) // 2048) * 2048

    src = edge_index[0]
    dst = edge_index[1]
    pad_e = E_PAD - e_real
    src_p = jnp.pad(src, (0, pad_e))
    dst_p = jnp.pad(dst, (0, pad_e))

    emask = jnp.pad(jnp.ones((e_real,), jnp.float32), (0, pad_e))

    gid_p = jnp.pad(graph_ids, (0, N_PAD - n))

    nmask = jnp.pad(jnp.ones((n,), jnp.float32), (0, N_PAD - n))

    W3p = jnp.pad(W3, ((0, 0), (0, 16 - hdim)))
    a3s = jnp.pad(a3_src, (0, 16 - hdim))
    a3d = jnp.pad(a3_dst, (0, 16 - hdim))

    zh1, sv1 = _gat_forward(x, W1, a1_src, a1_dst, src_p, dst_p,
                            emask, n, nt_pad, True)
    h1 = _tanh_bias2(zh1[:, :n, :], b1[None, :])
    zh2, sv2 = _gat_forward(h1, W2, a2_src, a2_dst, src_p, dst_p,
                            emask, n, nt_pad, True)
    h2 = _tanh_bias2(zh2[:, :n, :], b2[None, :])
    _, sv3 = _gat_forward(h2, W3p, a3s, a3d, src_p, dst_p,
                          emask, n, nt_pad, False)

    cnt_h = _scatter_scalar(nmask, gid_p, 128)
    invc = _inv_clip(cnt_h, 1.0)
    g3n = _gather_scalar(invc, gid_p)[:n]
    gout3 = _gout3_build(g3n[:, None], hdim)

    dhw3 = _gat_backward(gout3, W3p, a3s, a3d, sv3, src_p, dst_p, n, nt_pad)
    dz2 = _mm_tanh_bwd(dhw3, W3p.T, h2)
    dhw2 = _gat_backward(dz2, W2, a2_src, a2_dst, sv2, src_p, dst_p, n, nt_pad)
    dz1 = _mm_tanh_bwd(dhw2, W2.T, h1)
    dhw1 = _gat_backward(dz1, W1, a1_src, a1_dst, sv1, src_p, dst_p, n, nt_pad)
    dx = _mm_plain(dhw1, W1.T)
    return dx
